# baseline (speedup 1.0000x reference)
import functools

import jax
import jax.numpy as jnp
from jax import lax
from jax.experimental import pallas as pl
from jax.experimental.pallas import tpu as pltpu
from jax.experimental.pallas import tpu_sc as plsc

BR = 256


def _gather_call(user_idx, item_idx, user_table, item_table):
    b = user_idx.shape[0]
    d = user_table.shape[1]
    info = plsc.get_sparse_core_info()
    nw = info.num_cores * info.num_subcores
    b_per_w = b // nw
    mesh = plsc.VectorSubcoreMesh(core_axis_name="c", subcore_axis_name="s")

    @functools.partial(
        pl.kernel,
        mesh=mesh,
        out_type=[
            jax.ShapeDtypeStruct((b, d), jnp.float32),
            jax.ShapeDtypeStruct((b, d), jnp.float32),
        ],
        scratch_types=[
            pltpu.VMEM((b_per_w,), jnp.int32),
            pltpu.VMEM((b_per_w, d), jnp.float32),
            pltpu.VMEM((b_per_w,), jnp.int32),
            pltpu.VMEM((b_per_w, d), jnp.float32),
            pltpu.SemaphoreType.DMA,
            pltpu.SemaphoreType.DMA,
        ],
        compiler_params=pltpu.CompilerParams(use_tc_tiling_on_sc=False),
    )
    def _gather(uidx_hbm, iidx_hbm, utab_hbm, itab_hbm, uout_hbm, iout_hbm,
                uidx_v, urows_v, iidx_v, irows_v, usem, isem):
        wid = lax.axis_index("s") * info.num_cores + lax.axis_index("c")
        base = wid * b_per_w
        sl = pl.ds(base, b_per_w)
        pltpu.sync_copy(uidx_hbm.at[sl], uidx_v)
        pltpu.sync_copy(iidx_hbm.at[sl], iidx_v)
        ucp = pltpu.async_copy(utab_hbm.at[uidx_v], urows_v, usem)
        icp = pltpu.async_copy(itab_hbm.at[iidx_v], irows_v, isem)
        ucp.wait()
        pltpu.sync_copy(urows_v, uout_hbm.at[sl])
        icp.wait()
        pltpu.sync_copy(irows_v, iout_hbm.at[sl])

    return _gather(user_idx, item_idx, user_table, item_table)


def _prop_body(a_ref, u0_ref, i0_ref, out_ref,
               abf, u_cur, u_nxt, iT, iT_nxt, su, siT):
    r = pl.program_id(0)
    nblk = pl.num_programs(0)
    f32 = jnp.float32

    ab = a_ref[...].astype(jnp.bfloat16)
    abf[pl.ds(r * BR, BR), :] = ab

    u0b = u0_ref[...].astype(jnp.bfloat16)
    u_cur[pl.ds(r * BR, BR), :] = jnp.dot(ab, u0b, preferred_element_type=f32)

    i0b = i0_ref[pl.ds(r * BR, BR), :].astype(jnp.bfloat16)
    contrib = lax.dot_general(i0b, ab, (((0,), (0,)), ((), ())),
                              preferred_element_type=f32)

    @pl.when(r == 0)
    def _():
        iT[...] = contrib

    @pl.when(r != 0)
    def _():
        iT[...] = iT[...] + contrib

    @pl.when(r == nblk - 1)
    def _epilogue():
        b = u0_ref.shape[0]
        su[...] = u0_ref[...] + u_cur[...]
        siT[...] = iT[...]

        for _layer in range(2):
            ub = u_cur[...].astype(jnp.bfloat16)
            iT_nxt[...] = jnp.zeros_like(iT_nxt)

            def body(rr, carry):
                blk = pl.ds(rr * BR, BR)
                abb = abf[blk, :]
                u_nxt[blk, :] = jnp.dot(abb, ub, preferred_element_type=f32)
                itb = iT[:, blk].astype(jnp.bfloat16)
                iT_nxt[...] = iT_nxt[...] + jnp.dot(
                    itb, abb, preferred_element_type=f32)
                return carry

            lax.fori_loop(0, b // BR, body, 0)
            u_cur[...] = u_nxt[...]
            iT[...] = iT_nxt[...]
            su[...] = su[...] + u_nxt[...]
            siT[...] = siT[...] + iT_nxt[...]

        suv = su[...] * 0.25
        fi0 = i0_ref[...] * 0.25
        term1 = jnp.sum(suv * fi0, axis=1, keepdims=True)
        siv = (siT[...] * 0.25).astype(jnp.bfloat16)
        sub = suv.astype(jnp.bfloat16)
        for nb in range(b // BR):
            p = jnp.dot(sub[nb * BR:(nb + 1) * BR, :],
                        siv[:, nb * BR:(nb + 1) * BR],
                        preferred_element_type=f32)
            rows = lax.broadcasted_iota(jnp.int32, (BR, BR), 0)
            cols = lax.broadcasted_iota(jnp.int32, (BR, BR), 1)
            term2 = jnp.sum(jnp.where(rows == cols, p, 0.0),
                            axis=1, keepdims=True)
            scores = term1[nb * BR:(nb + 1) * BR, :] + term2
            out_ref[pl.ds(nb * BR, BR), :] = jax.nn.sigmoid(scores)


def _prop_call(adj, u0, i0):
    b, d = u0.shape
    nblk = b // BR
    return pl.pallas_call(
        _prop_body,
        grid=(nblk,),
        in_specs=[
            pl.BlockSpec((BR, b), lambda r: (r, 0)),
            pl.BlockSpec((b, d), lambda r: (0, 0)),
            pl.BlockSpec((b, d), lambda r: (0, 0)),
        ],
        out_specs=pl.BlockSpec((b, 1), lambda r: (0, 0)),
        out_shape=jax.ShapeDtypeStruct((b, 1), jnp.float32),
        scratch_shapes=[
            pltpu.VMEM((b, b), jnp.bfloat16),
            pltpu.VMEM((b, d), jnp.float32),
            pltpu.VMEM((b, d), jnp.float32),
            pltpu.VMEM((d, b), jnp.float32),
            pltpu.VMEM((d, b), jnp.float32),
            pltpu.VMEM((b, d), jnp.float32),
            pltpu.VMEM((d, b), jnp.float32),
        ],
        compiler_params=pltpu.CompilerParams(
            vmem_limit_bytes=100 * 1024 * 1024,
        ),
    )(adj, u0, i0)


def kernel(user_indices, item_indices, adj_matrix, user_table, item_table):
    ui = user_indices.astype(jnp.int32)
    ii = item_indices.astype(jnp.int32)
    u0, i0 = _gather_call(ui, ii, user_table, item_table)
    preds = _prop_call(adj_matrix, u0, i0)
    return preds.reshape(user_indices.shape[0])


# speedup vs baseline: 1.1008x; 1.1008x over previous
import functools

import jax
import jax.numpy as jnp
from jax import lax
from jax.experimental import pallas as pl
from jax.experimental.pallas import tpu as pltpu
from jax.experimental.pallas import tpu_sc as plsc

BR = 256


def _gather_call(user_idx, item_idx, both_tables):
    b = user_idx.shape[0]
    dd = both_tables.shape[1]
    info = plsc.get_sparse_core_info()
    nw = info.num_cores * info.num_subcores
    b_per_w = b // nw
    mesh = plsc.VectorSubcoreMesh(core_axis_name="c", subcore_axis_name="s")

    @functools.partial(
        pl.kernel,
        mesh=mesh,
        out_type=[
            jax.ShapeDtypeStruct((b, dd), jnp.float32),
            jax.ShapeDtypeStruct((b, dd), jnp.float32),
        ],
        scratch_types=[
            pltpu.VMEM((b_per_w,), jnp.int32),
            pltpu.VMEM((b_per_w, dd), jnp.float32),
            pltpu.VMEM((b_per_w,), jnp.int32),
            pltpu.VMEM((b_per_w, dd), jnp.float32),
            pltpu.SemaphoreType.DMA,
            pltpu.SemaphoreType.DMA,
        ],
    )
    def _gather(uidx_hbm, iidx_hbm, tab_hbm, uout_hbm, iout_hbm,
                uidx_v, urows_v, iidx_v, irows_v, usem, isem):
        wid = lax.axis_index("s") * info.num_cores + lax.axis_index("c")
        base = wid * b_per_w
        sl = pl.ds(base, b_per_w)
        pltpu.sync_copy(uidx_hbm.at[sl], uidx_v)
        pltpu.sync_copy(iidx_hbm.at[sl], iidx_v)
        ucp = pltpu.async_copy(tab_hbm.at[uidx_v], urows_v, usem)
        icp = pltpu.async_copy(tab_hbm.at[iidx_v], irows_v, isem)
        ucp.wait()
        pltpu.sync_copy(urows_v, uout_hbm.at[sl])
        icp.wait()
        pltpu.sync_copy(irows_v, iout_hbm.at[sl])

    return _gather(user_idx, item_idx, both_tables)


def _prop_body(a_ref, u0_ref, i0_ref, out_ref,
               abf, u_cur, u_nxt, iT, iT_nxt, su, siT):
    r = pl.program_id(0)
    nblk = pl.num_programs(0)
    f32 = jnp.float32

    ab = a_ref[...].astype(jnp.bfloat16)
    abf[pl.ds(r * BR, BR), :] = ab

    u0b = u0_ref[...].astype(jnp.bfloat16)
    u_cur[pl.ds(r * BR, BR), :] = jnp.dot(ab, u0b, preferred_element_type=f32)

    i0b = i0_ref[pl.ds(r * BR, BR), :].astype(jnp.bfloat16)
    contrib = lax.dot_general(i0b, ab, (((0,), (0,)), ((), ())),
                              preferred_element_type=f32)

    @pl.when(r == 0)
    def _():
        iT[...] = contrib

    @pl.when(r != 0)
    def _():
        iT[...] = iT[...] + contrib

    @pl.when(r == nblk - 1)
    def _epilogue():
        b = u0_ref.shape[0]
        su[...] = u0_ref[...] + u_cur[...]
        siT[...] = iT[...]

        for _layer in range(2):
            ub = u_cur[...].astype(jnp.bfloat16)
            iT_nxt[...] = jnp.zeros_like(iT_nxt)

            def body(rr, carry):
                blk = pl.ds(rr * BR, BR)
                abb = abf[blk, :]
                u_nxt[blk, :] = jnp.dot(abb, ub, preferred_element_type=f32)
                itb = iT[:, blk].astype(jnp.bfloat16)
                iT_nxt[...] = iT_nxt[...] + jnp.dot(
                    itb, abb, preferred_element_type=f32)
                return carry

            lax.fori_loop(0, b // BR, body, 0)
            u_cur[...] = u_nxt[...]
            iT[...] = iT_nxt[...]
            su[...] = su[...] + u_nxt[...]
            siT[...] = siT[...] + iT_nxt[...]

        suv = su[...] * 0.25
        fi0 = i0_ref[...] * 0.25
        term1 = jnp.sum(suv * fi0, axis=1, keepdims=True)
        siv = (siT[...] * 0.25).astype(jnp.bfloat16)
        sub = suv.astype(jnp.bfloat16)
        for nb in range(b // BR):
            p = jnp.dot(sub[nb * BR:(nb + 1) * BR, :],
                        siv[:, nb * BR:(nb + 1) * BR],
                        preferred_element_type=f32)
            rows = lax.broadcasted_iota(jnp.int32, (BR, BR), 0)
            cols = lax.broadcasted_iota(jnp.int32, (BR, BR), 1)
            term2 = jnp.sum(jnp.where(rows == cols, p, 0.0),
                            axis=1, keepdims=True)
            scores = term1[nb * BR:(nb + 1) * BR, :] + term2
            out_ref[pl.ds(nb * BR, BR), :] = jax.nn.sigmoid(scores)


def _prop_call(adj, u0, i0):
    b, d = u0.shape
    nblk = b // BR
    return pl.pallas_call(
        _prop_body,
        grid=(nblk,),
        in_specs=[
            pl.BlockSpec((BR, b), lambda r: (r, 0)),
            pl.BlockSpec((b, d), lambda r: (0, 0)),
            pl.BlockSpec((b, d), lambda r: (0, 0)),
        ],
        out_specs=pl.BlockSpec((b, 1), lambda r: (0, 0)),
        out_shape=jax.ShapeDtypeStruct((b, 1), jnp.float32),
        scratch_shapes=[
            pltpu.VMEM((b, b), jnp.bfloat16),
            pltpu.VMEM((b, d), jnp.float32),
            pltpu.VMEM((b, d), jnp.float32),
            pltpu.VMEM((d, b), jnp.float32),
            pltpu.VMEM((d, b), jnp.float32),
            pltpu.VMEM((b, d), jnp.float32),
            pltpu.VMEM((d, b), jnp.float32),
        ],
        compiler_params=pltpu.CompilerParams(
            vmem_limit_bytes=100 * 1024 * 1024,
        ),
    )(adj, u0, i0)


def kernel(user_indices, item_indices, adj_matrix, user_table, item_table):
    ui = user_indices.astype(jnp.int32)
    ii = item_indices.astype(jnp.int32)
    d = user_table.shape[1]
    tab = jnp.concatenate([user_table, item_table], axis=1)
    gu, gi = _gather_call(ui, ii, tab)
    u0 = gu[:, :d]
    i0 = gi[:, d:]
    preds = _prop_call(adj_matrix, u0, i0)
    return preds.reshape(user_indices.shape[0])


# speedup vs baseline: 1.4591x; 1.3255x over previous
import functools

import jax
import jax.numpy as jnp
from jax import lax
from jax.experimental import pallas as pl
from jax.experimental.pallas import tpu as pltpu
from jax.experimental.pallas import tpu_sc as plsc

BR = 256


CB = 4096


def _concat_body(ut_ref, it_ref, out_ref):
    ub = lax.transpose(ut_ref[...], (1, 0))
    ib = lax.transpose(it_ref[...], (1, 0))
    out_ref[...] = jnp.concatenate([ub, ib], axis=1)


def _concat_tables(ut, it):
    d, n = ut.shape
    nblk = (n + CB - 1) // CB
    return pl.pallas_call(
        _concat_body,
        grid=(nblk,),
        in_specs=[
            pl.BlockSpec((d, CB), lambda c: (0, c)),
            pl.BlockSpec((d, CB), lambda c: (0, c)),
        ],
        out_specs=pl.BlockSpec((CB, 2 * d), lambda c: (c, 0)),
        out_shape=jax.ShapeDtypeStruct((n, 2 * d), jnp.float32),
    )(ut, it)


def _gather_call(user_idx, item_idx, both_tables):
    b = user_idx.shape[0]
    dd = both_tables.shape[1]
    info = plsc.get_sparse_core_info()
    nw = info.num_cores * info.num_subcores
    b_per_w = b // nw
    mesh = plsc.VectorSubcoreMesh(core_axis_name="c", subcore_axis_name="s")

    @functools.partial(
        pl.kernel,
        mesh=mesh,
        out_type=[
            jax.ShapeDtypeStruct((b, dd), jnp.float32),
            jax.ShapeDtypeStruct((b, dd), jnp.float32),
        ],
        scratch_types=[
            pltpu.VMEM((b_per_w,), jnp.int32),
            pltpu.VMEM((b_per_w, dd), jnp.float32),
            pltpu.VMEM((b_per_w,), jnp.int32),
            pltpu.VMEM((b_per_w, dd), jnp.float32),
            pltpu.SemaphoreType.DMA,
            pltpu.SemaphoreType.DMA,
        ],
    )
    def _gather(uidx_hbm, iidx_hbm, tab_hbm, uout_hbm, iout_hbm,
                uidx_v, urows_v, iidx_v, irows_v, usem, isem):
        wid = lax.axis_index("s") * info.num_cores + lax.axis_index("c")
        base = wid * b_per_w
        sl = pl.ds(base, b_per_w)
        pltpu.sync_copy(uidx_hbm.at[sl], uidx_v)
        pltpu.sync_copy(iidx_hbm.at[sl], iidx_v)
        ucp = pltpu.async_copy(tab_hbm.at[uidx_v], urows_v, usem)
        icp = pltpu.async_copy(tab_hbm.at[iidx_v], irows_v, isem)
        ucp.wait()
        pltpu.sync_copy(urows_v, uout_hbm.at[sl])
        icp.wait()
        pltpu.sync_copy(irows_v, iout_hbm.at[sl])

    return _gather(user_idx, item_idx, both_tables)


def _prop_body(a_ref, u0_ref, i0_ref, out_ref,
               abf, u_cur, u_nxt, iT, iT_nxt, su, siT):
    r = pl.program_id(0)
    nblk = pl.num_programs(0)
    f32 = jnp.float32

    d = u0_ref.shape[1] // 2

    ab = a_ref[...].astype(jnp.bfloat16)
    abf[pl.ds(r * BR, BR), :] = ab

    u0b = u0_ref[:, :d].astype(jnp.bfloat16)
    u_cur[pl.ds(r * BR, BR), :] = jnp.dot(ab, u0b, preferred_element_type=f32)

    i0b = i0_ref[pl.ds(r * BR, BR), d:].astype(jnp.bfloat16)
    contrib = lax.dot_general(i0b, ab, (((0,), (0,)), ((), ())),
                              preferred_element_type=f32)

    @pl.when(r == 0)
    def _():
        iT[...] = contrib

    @pl.when(r != 0)
    def _():
        iT[...] = iT[...] + contrib

    @pl.when(r == nblk - 1)
    def _epilogue():
        b = u0_ref.shape[0]
        su[...] = u0_ref[:, :d] + u_cur[...]
        siT[...] = iT[...]

        for _layer in range(2):
            ub = u_cur[...].astype(jnp.bfloat16)
            iT_nxt[...] = jnp.zeros_like(iT_nxt)

            def body(rr, carry):
                blk = pl.ds(rr * BR, BR)
                abb = abf[blk, :]
                u_nxt[blk, :] = jnp.dot(abb, ub, preferred_element_type=f32)
                itb = iT[:, blk].astype(jnp.bfloat16)
                iT_nxt[...] = iT_nxt[...] + jnp.dot(
                    itb, abb, preferred_element_type=f32)
                return carry

            lax.fori_loop(0, b // BR, body, 0)
            u_cur[...] = u_nxt[...]
            iT[...] = iT_nxt[...]
            su[...] = su[...] + u_nxt[...]
            siT[...] = siT[...] + iT_nxt[...]

        suv = su[...] * 0.25
        fi0 = i0_ref[:, d:] * 0.25
        term1 = jnp.sum(suv * fi0, axis=1, keepdims=True)
        siv = (siT[...] * 0.25).astype(jnp.bfloat16)
        sub = suv.astype(jnp.bfloat16)
        for nb in range(b // BR):
            p = jnp.dot(sub[nb * BR:(nb + 1) * BR, :],
                        siv[:, nb * BR:(nb + 1) * BR],
                        preferred_element_type=f32)
            rows = lax.broadcasted_iota(jnp.int32, (BR, BR), 0)
            cols = lax.broadcasted_iota(jnp.int32, (BR, BR), 1)
            term2 = jnp.sum(jnp.where(rows == cols, p, 0.0),
                            axis=1, keepdims=True)
            scores = term1[nb * BR:(nb + 1) * BR, :] + term2
            out_ref[pl.ds(nb * BR, BR), :] = jax.nn.sigmoid(scores)


def _prop_call(adj, gu, gi):
    b, dd = gu.shape
    d = dd // 2
    nblk = b // BR
    return pl.pallas_call(
        _prop_body,
        grid=(nblk,),
        in_specs=[
            pl.BlockSpec((BR, b), lambda r: (r, 0)),
            pl.BlockSpec((b, dd), lambda r: (0, 0)),
            pl.BlockSpec((b, dd), lambda r: (0, 0)),
        ],
        out_specs=pl.BlockSpec((b, 1), lambda r: (0, 0)),
        out_shape=jax.ShapeDtypeStruct((b, 1), jnp.float32),
        scratch_shapes=[
            pltpu.VMEM((b, b), jnp.bfloat16),
            pltpu.VMEM((b, d), jnp.float32),
            pltpu.VMEM((b, d), jnp.float32),
            pltpu.VMEM((d, b), jnp.float32),
            pltpu.VMEM((d, b), jnp.float32),
            pltpu.VMEM((b, d), jnp.float32),
            pltpu.VMEM((d, b), jnp.float32),
        ],
        compiler_params=pltpu.CompilerParams(
            vmem_limit_bytes=100 * 1024 * 1024,
        ),
    )(adj, gu, gi)


def kernel(user_indices, item_indices, adj_matrix, user_table, item_table):
    ui = user_indices.astype(jnp.int32)
    ii = item_indices.astype(jnp.int32)
    tab = _concat_tables(user_table.T, item_table.T)
    gu, gi = _gather_call(ui, ii, tab)
    preds = _prop_call(adj_matrix, gu, gi)
    return preds.reshape(user_indices.shape[0])
